# Initial kernel scaffold; baseline (speedup 1.0000x reference)
import jax
import jax.numpy as jnp
from jax.experimental import pallas as pl


def kernel(x, vote_6d, scale, vote_presence_logit, batch):
    raise NotImplementedError("write your pallas kernel here")


# speedup vs baseline: 32.3216x; 32.3216x over previous
import functools
import math

import jax
import jax.numpy as jnp
from jax.experimental import pallas as pl
from jax.experimental.pallas import tpu as pltpu

N = 32768
B = 16
NCV = 128
D = 6
EPS = 1e-10
BLK = 512
GRID = N // BLK
_HALF_LOG_2PI = 0.5 * math.log(2.0 * math.pi)


def _body(x_ref, votes_ref, scale_ref, logit_ref, batch_ref, seg_ref, mean_ref):
    i = pl.program_id(0)

    s = jnp.maximum(scale_ref[...], EPS)
    inv2 = 1.0 / (s * s)
    ls = logit_ref[...] - D * jnp.log(s) - D * _HALF_LOG_2PI
    tab = jnp.concatenate(
        [votes_ref[d] for d in range(D)] + [inv2, ls], axis=1)

    bids = batch_ref[...]
    cols = jax.lax.broadcasted_iota(jnp.int32, (BLK, B), 1)
    oh = jnp.where(bids == cols, 1.0, 0.0).astype(jnp.float32)

    g = jax.lax.dot(oh, tab, preferred_element_type=jnp.float32)

    x = x_ref[...]
    acc = jnp.zeros((BLK, NCV), jnp.float32)
    for d in range(D):
        diff = x[:, d:d + 1] - g[:, d * NCV:(d + 1) * NCV]
        acc = acc + diff * diff
    post = g[:, (D + 1) * NCV:] - 0.5 * acc * g[:, D * NCV:(D + 1) * NCV]

    mx = jnp.max(post, axis=1, keepdims=True)
    lpp = mx + jnp.log(jnp.sum(jnp.exp(post - mx), axis=1, keepdims=True))

    seg_part = jnp.sum(oh * lpp, axis=0)

    @pl.when(i == 0)
    def _():
        seg_ref[...] = jnp.zeros_like(seg_ref)

    seg_ref[...] += seg_part[None, :]

    @pl.when(i == GRID - 1)
    def _():
        mean_ref[...] = jnp.sum(seg_ref[...], keepdims=True) * (1.0 / B)


@jax.jit
def kernel(x, vote_6d, scale, vote_presence_logit, batch):
    votes_t = jnp.transpose(vote_6d.reshape(B, NCV, D), (2, 0, 1))
    scale_r = scale.reshape(B, NCV)
    logit_r = vote_presence_logit.reshape(B, NCV)
    batch_c = batch.reshape(N, 1)

    seg2d, mean2d = pl.pallas_call(
        _body,
        grid=(GRID,),
        in_specs=[
            pl.BlockSpec((BLK, D), lambda i: (i, 0)),
            pl.BlockSpec((D, B, NCV), lambda i: (0, 0, 0)),
            pl.BlockSpec((B, NCV), lambda i: (0, 0)),
            pl.BlockSpec((B, NCV), lambda i: (0, 0)),
            pl.BlockSpec((BLK, 1), lambda i: (i, 0)),
        ],
        out_specs=[
            pl.BlockSpec((1, 16), lambda i: (0, 0)),
            pl.BlockSpec((1, 1), lambda i: (0, 0)),
        ],
        out_shape=[
            jax.ShapeDtypeStruct((1, B), jnp.float32),
            jax.ShapeDtypeStruct((1, 1), jnp.float32),
        ],
        compiler_params=pltpu.CompilerParams(
            dimension_semantics=("arbitrary",)),
    )(x, votes_t, scale_r, logit_r, batch_c)
    return (mean2d.reshape(()), seg2d.reshape(B))
